# baseline (speedup 1.0000x reference)
import functools

import jax
import jax.numpy as jnp
from jax import lax
from jax.experimental import pallas as pl
from jax.experimental.pallas import tpu as pltpu
from jax.experimental.pallas import tpu_sc as plsc

_N = 10000
_E = 320000
_D = 128
_H = 64
_B = 8192

_W1 = 80
_NC = 2
_NS = 16
_NW = _NC * _NS
_CH = 128
_NCHK = 80
_EP = _NW * _NCHK * _CH
_NP = 10240
_RPT = _NP // _NS
_BPW = _B // _NW

_mesh = plsc.VectorSubcoreMesh(core_axis_name="c", subcore_axis_name="s")


_NBUF = 5


def _make_segsum(width):
    @functools.partial(
        pl.kernel,
        out_type=jax.ShapeDtypeStruct((_NC, _NP, width), jnp.float32),
        mesh=_mesh,
        compiler_params=pltpu.CompilerParams(use_tc_tiling_on_sc=False),
        scratch_types=[
            pltpu.VMEM((_NCHK, _CH), jnp.int32),
            pltpu.VMEM((_NCHK, _CH), jnp.int32),
            pltpu.VMEM((_NBUF, _CH, width), jnp.float32),
            pltpu.VMEM_SHARED((_NP, width), jnp.float32),
            pltpu.SemaphoreType.DMA((_NBUF,)),
            pltpu.SemaphoreType.DMA((_NBUF,)),
        ],
    )
    def seg(s_hbm, d_hbm, tab_hbm, zeros_hbm, out_hbm,
            sidx_v, didx_v, rows_v, acc_sh, gsem, ssem):
        cid = lax.axis_index("c")
        sid = lax.axis_index("s")
        wid = cid * _NS + sid
        pltpu.sync_copy(zeros_hbm.at[pl.ds(sid * _RPT, _RPT)],
                        acc_sh.at[pl.ds(sid * _RPT, _RPT)])
        pltpu.sync_copy(s_hbm.at[wid], sidx_v)
        pltpu.sync_copy(d_hbm.at[wid], didx_v)
        plsc.subcore_barrier()

        for b in range(_NBUF):
            pltpu.async_copy(tab_hbm.at[sidx_v.at[b]], rows_v.at[b], gsem.at[b])

        def outer(t, carry):
            j0 = t * _NBUF
            for b in range(_NBUF):
                pltpu.make_async_copy(
                    tab_hbm.at[sidx_v.at[j0 + b]], rows_v.at[b], gsem.at[b]).wait()
                pltpu.async_copy(
                    rows_v.at[b], acc_sh.at[didx_v.at[j0 + b]], ssem.at[b], add=True)
            for b in range(_NBUF):
                nj = j0 + _NBUF + b

                @pl.when(nj < _NCHK)
                def _():
                    pltpu.make_async_copy(
                        rows_v.at[b], acc_sh.at[didx_v.at[j0 + b]], ssem.at[b]).wait()
                    pltpu.async_copy(
                        tab_hbm.at[sidx_v.at[nj]], rows_v.at[b], gsem.at[b])
            return carry

        lax.fori_loop(0, _NCHK // _NBUF, outer, 0)
        jlast = _NCHK - _NBUF
        for b in range(_NBUF):
            pltpu.make_async_copy(
                rows_v.at[b], acc_sh.at[didx_v.at[jlast + b]], ssem.at[b]).wait()
        plsc.subcore_barrier()
        pltpu.sync_copy(acc_sh.at[pl.ds(sid * _RPT, _RPT)],
                        out_hbm.at[cid, pl.ds(sid * _RPT, _RPT)])

    return seg


_segsum80 = _make_segsum(_W1)
_segsum64 = _make_segsum(_H)


@functools.partial(
    pl.kernel,
    out_type=jax.ShapeDtypeStruct((_B,), jnp.float32),
    mesh=_mesh,
    compiler_params=pltpu.CompilerParams(
        use_tc_tiling_on_sc=False, needs_layout_passes=False),
    scratch_types=[
        pltpu.VMEM((2, 128), jnp.int32),
        pltpu.VMEM((2, 128), jnp.int32),
        pltpu.VMEM((_BPW, _H), jnp.float32),
        pltpu.VMEM((_BPW, _H), jnp.float32),
        pltpu.VMEM((_BPW,), jnp.float32),
        pltpu.SemaphoreType.DMA((4,)),
    ],
)
def _decode(si_hbm, di_hbm, z_hbm, out_hbm, si_v, di_v, zs_v, zd_v, o_v, sems):
    cid = lax.axis_index("c")
    sid = lax.axis_index("s")
    wid = cid * _NS + sid
    pltpu.sync_copy(si_hbm.at[wid], si_v)
    pltpu.sync_copy(di_hbm.at[wid], di_v)
    for t in range(2):
        pltpu.async_copy(z_hbm.at[si_v.at[t]], zs_v.at[pl.ds(t * 128, 128)], sems.at[t])
        pltpu.async_copy(z_hbm.at[di_v.at[t]], zd_v.at[pl.ds(t * 128, 128)], sems.at[2 + t])
    for t in range(2):
        pltpu.make_async_copy(
            z_hbm.at[si_v.at[t]], zs_v.at[pl.ds(t * 128, 128)], sems.at[t]).wait()
        pltpu.make_async_copy(
            z_hbm.at[di_v.at[t]], zd_v.at[pl.ds(t * 128, 128)], sems.at[2 + t]).wait()

    lanes = lax.iota(jnp.int32, 16)

    def body(i, acc):
        r = lax.shift_right_logical(i, 6)
        c = lax.bitwise_and(i, 63)
        rows16 = r * 16 + lanes
        cc = jnp.zeros((16,), jnp.int32) + c
        vs = plsc.load_gather(zs_v, [rows16, cc])
        vd = plsc.load_gather(zd_v, [rows16, cc])
        acc = acc * jnp.where(c == 0, 0.0, 1.0) + vs * vd

        @pl.when(c == _H - 1)
        def _():
            o_v[pl.ds(r * 16, 16)] = 1.0 / (1.0 + jnp.exp(-acc))

        return acc

    lax.fori_loop(0, (_BPW // 16) * _H, body, jnp.zeros((16,), jnp.float32))
    pltpu.sync_copy(o_v, out_hbm.at[pl.ds(wid * _BPW, _BPW)])


_CT = (((1,), (1,)), ((), ()))


def _tc_dense1(x, W1le, e1, W1r, b1):
    g = 10
    bn = _N // g

    def body(x_ref, wle_ref, e1_ref, wr_ref, b1_ref, yext_ref, r1_ref):
        xb = x_ref[...]
        yext_ref[...] = lax.dot_general(
            xb, wle_ref[...], _CT, preferred_element_type=jnp.float32) + e1_ref[...]
        r1_ref[...] = lax.dot_general(
            xb, wr_ref[...], _CT, preferred_element_type=jnp.float32) + b1_ref[...]

    return pl.pallas_call(
        body,
        grid=(g,),
        in_specs=[pl.BlockSpec((bn, _D), lambda i: (i, 0)),
                  pl.BlockSpec((_W1, _D), lambda i: (0, 0)),
                  pl.BlockSpec((1, _W1), lambda i: (0, 0)),
                  pl.BlockSpec((_H, _D), lambda i: (0, 0)),
                  pl.BlockSpec((1, _H), lambda i: (0, 0))],
        out_specs=[pl.BlockSpec((bn, _W1), lambda i: (i, 0)),
                   pl.BlockSpec((bn, _H), lambda i: (i, 0))],
        out_shape=[jax.ShapeDtypeStruct((_N, _W1), jnp.float32),
                   jax.ShapeDtypeStruct((_N, _H), jnp.float32)],
    )(x, W1le, e1, W1r, b1)


def _tc_dense2(agg1p, r1, W2l, b2, W2r):
    g = 10
    bn = _N // g

    def body(aggp_ref, r1_ref, wl_ref, b_ref, wr_ref, y2_ref, r2_ref, inv_ref):
        a = aggp_ref[...]
        agg = a[0] + a[1]
        inv = 1.0 / jnp.maximum(agg[:, _H:_H + 1], 1.0)
        h = jnp.maximum(agg[:, :_H] * inv + r1_ref[...], 0.0)
        y2_ref[...] = lax.dot_general(
            h, wl_ref[...], _CT, preferred_element_type=jnp.float32)
        r2_ref[...] = lax.dot_general(
            h, wr_ref[...], _CT, preferred_element_type=jnp.float32) + b_ref[...]
        inv_ref[...] = inv

    return pl.pallas_call(
        body,
        grid=(g,),
        in_specs=[pl.BlockSpec((_NC, bn, _W1), lambda i: (0, i, 0)),
                  pl.BlockSpec((bn, _H), lambda i: (i, 0)),
                  pl.BlockSpec((_H, _H), lambda i: (0, 0)),
                  pl.BlockSpec((1, _H), lambda i: (0, 0)),
                  pl.BlockSpec((_H, _H), lambda i: (0, 0))],
        out_specs=[pl.BlockSpec((bn, _H), lambda i: (i, 0)),
                   pl.BlockSpec((bn, _H), lambda i: (i, 0)),
                   pl.BlockSpec((bn, 1), lambda i: (i, 0))],
        out_shape=[jax.ShapeDtypeStruct((_N, _H), jnp.float32),
                   jax.ShapeDtypeStruct((_N, _H), jnp.float32),
                   jax.ShapeDtypeStruct((_N, 1), jnp.float32)],
    )(agg1p, r1, W2l, b2, W2r)


def _tc_dense3(agg2p, r2, inv):
    g = 10
    bn = _N // g

    def body(aggp_ref, r2_ref, inv_ref, z_ref):
        a = aggp_ref[...]
        z_ref[...] = jnp.maximum((a[0] + a[1]) * inv_ref[...] + r2_ref[...], 0.0)

    return pl.pallas_call(
        body,
        grid=(g,),
        in_specs=[pl.BlockSpec((_NC, bn, _H), lambda i: (0, i, 0)),
                  pl.BlockSpec((bn, _H), lambda i: (i, 0)),
                  pl.BlockSpec((bn, 1), lambda i: (i, 0))],
        out_specs=pl.BlockSpec((bn, _H), lambda i: (i, 0)),
        out_shape=jax.ShapeDtypeStruct((_N, _H), jnp.float32),
    )(agg2p, r2, inv)


def kernel(x, edge_index, src, dst, W1l, b1l, W1r, W2l, b2l, W2r):
    f32 = jnp.float32
    W1le = jnp.zeros((_W1, _D), f32).at[:_H].set(W1l)
    e1 = jnp.zeros((1, _W1), f32).at[0, _H].set(1.0)

    pad = _EP - _E
    s_r = jnp.concatenate(
        [edge_index[0], jnp.arange(pad, dtype=jnp.int32) % _N]
    ).reshape(_NW, _NCHK, _CH)
    d_r = jnp.concatenate(
        [edge_index[1], _N + jnp.arange(pad, dtype=jnp.int32) % (_NP - _N)]
    ).reshape(_NW, _NCHK, _CH)
    zeros1 = jnp.zeros((_NP, _W1), f32)
    zeros2 = jnp.zeros((_NP, _H), f32)

    yext, r1 = _tc_dense1(x, W1le, e1, W1r, b1l.reshape(1, _H))
    agg1p = _segsum80(s_r, d_r, yext, zeros1)
    y2, r2, inv = _tc_dense2(agg1p, r1, W2l, b2l.reshape(1, _H), W2r)
    agg2p = _segsum64(s_r, d_r, y2, zeros2)
    z = _tc_dense3(agg2p, r2, inv)

    return _decode(src.reshape(_NW, 2, 128), dst.reshape(_NW, 2, 128), z)


# speedup vs baseline: 1.1337x; 1.1337x over previous
import functools

import jax
import jax.numpy as jnp
from jax import lax
from jax.experimental import pallas as pl
from jax.experimental.pallas import tpu as pltpu
from jax.experimental.pallas import tpu_sc as plsc

_N = 10000
_E = 320000
_D = 128
_H = 64
_B = 8192

_W1 = 80
_NC = 2
_NS = 16
_NW = _NC * _NS
_CH = 128
_NCHK = 80
_EP = _NW * _NCHK * _CH
_NP = 10240
_RPT = _NP // _NS
_BPW = _B // _NW

_mesh = plsc.VectorSubcoreMesh(core_axis_name="c", subcore_axis_name="s")


_NBUF = 5


def _make_segsum(width):
    @functools.partial(
        pl.kernel,
        out_type=jax.ShapeDtypeStruct((_NC, _NP, width), jnp.float32),
        mesh=_mesh,
        compiler_params=pltpu.CompilerParams(use_tc_tiling_on_sc=False),
        scratch_types=[
            pltpu.VMEM((_NCHK, _CH), jnp.int32),
            pltpu.VMEM((_NCHK, _CH), jnp.int32),
            pltpu.VMEM((_NBUF, _CH, width), jnp.float32),
            pltpu.VMEM_SHARED((_NP, width), jnp.float32),
            pltpu.SemaphoreType.DMA((_NBUF,)),
            pltpu.SemaphoreType.DMA((_NBUF,)),
        ],
    )
    def seg(s_hbm, d_hbm, tab_hbm, zeros_hbm, out_hbm,
            sidx_v, didx_v, rows_v, acc_sh, gsem, ssem):
        cid = lax.axis_index("c")
        sid = lax.axis_index("s")
        wid = cid * _NS + sid
        pltpu.sync_copy(zeros_hbm.at[pl.ds(sid * _RPT, _RPT)],
                        acc_sh.at[pl.ds(sid * _RPT, _RPT)])
        pltpu.sync_copy(s_hbm.at[wid], sidx_v)
        pltpu.sync_copy(d_hbm.at[wid], didx_v)
        plsc.subcore_barrier()

        for b in range(_NBUF):
            pltpu.async_copy(tab_hbm.at[sidx_v.at[b]], rows_v.at[b], gsem.at[b])

        def outer(t, carry):
            j0 = t * _NBUF
            for b in range(_NBUF):
                pltpu.make_async_copy(
                    tab_hbm.at[sidx_v.at[j0 + b]], rows_v.at[b], gsem.at[b]).wait()
                pltpu.async_copy(
                    rows_v.at[b], acc_sh.at[didx_v.at[j0 + b]], ssem.at[b], add=True)
            for b in range(_NBUF):
                nj = j0 + _NBUF + b

                @pl.when(nj < _NCHK)
                def _():
                    pltpu.make_async_copy(
                        rows_v.at[b], acc_sh.at[didx_v.at[j0 + b]], ssem.at[b]).wait()
                    pltpu.async_copy(
                        tab_hbm.at[sidx_v.at[nj]], rows_v.at[b], gsem.at[b])
            return carry

        lax.fori_loop(0, _NCHK // _NBUF, outer, 0)
        jlast = _NCHK - _NBUF
        for b in range(_NBUF):
            pltpu.make_async_copy(
                rows_v.at[b], acc_sh.at[didx_v.at[jlast + b]], ssem.at[b]).wait()
        plsc.subcore_barrier()
        pltpu.sync_copy(acc_sh.at[pl.ds(sid * _RPT, _RPT)],
                        out_hbm.at[cid, pl.ds(sid * _RPT, _RPT)])

    return seg


_segsum80 = _make_segsum(_W1)
_segsum64 = _make_segsum(_H)


@functools.partial(
    pl.kernel,
    out_type=jax.ShapeDtypeStruct((_B,), jnp.float32),
    mesh=_mesh,
    compiler_params=pltpu.CompilerParams(
        use_tc_tiling_on_sc=False, needs_layout_passes=False),
    scratch_types=[
        pltpu.VMEM((2, 128), jnp.int32),
        pltpu.VMEM((2, 128), jnp.int32),
        pltpu.VMEM((_BPW, _H), jnp.float32),
        pltpu.VMEM((_BPW, _H), jnp.float32),
        pltpu.VMEM((_BPW, 16), jnp.float32),
        pltpu.VMEM((_BPW,), jnp.float32),
        pltpu.SemaphoreType.DMA((4,)),
    ],
)
def _decode(si_hbm, di_hbm, z_hbm, out_hbm, si_v, di_v, zs_v, zd_v, stage_v, o_v,
            sems):
    cid = lax.axis_index("c")
    sid = lax.axis_index("s")
    wid = cid * _NS + sid
    pltpu.sync_copy(si_hbm.at[wid], si_v)
    pltpu.sync_copy(di_hbm.at[wid], di_v)
    for t in range(2):
        pltpu.async_copy(z_hbm.at[si_v.at[t]], zs_v.at[pl.ds(t * 128, 128)], sems.at[t])
        pltpu.async_copy(z_hbm.at[di_v.at[t]], zd_v.at[pl.ds(t * 128, 128)], sems.at[2 + t])
    for t in range(2):
        pltpu.make_async_copy(
            z_hbm.at[si_v.at[t]], zs_v.at[pl.ds(t * 128, 128)], sems.at[t]).wait()
        pltpu.make_async_copy(
            z_hbm.at[di_v.at[t]], zd_v.at[pl.ds(t * 128, 128)], sems.at[2 + t]).wait()

    def body(p, carry):
        t = ((zs_v[p, pl.ds(0, 16)] * zd_v[p, pl.ds(0, 16)]
              + zs_v[p, pl.ds(16, 16)] * zd_v[p, pl.ds(16, 16)])
             + (zs_v[p, pl.ds(32, 16)] * zd_v[p, pl.ds(32, 16)]
                + zs_v[p, pl.ds(48, 16)] * zd_v[p, pl.ds(48, 16)]))
        stage_v[p, pl.ds(0, 16)] = jnp.cumsum(t)
        return carry

    lax.fori_loop(0, _BPW, body, 0)

    lanes = lax.iota(jnp.int32, 16)
    c15 = jnp.zeros((16,), jnp.int32) + 15

    def sig(g, carry):
        v = plsc.load_gather(stage_v, [g * 16 + lanes, c15])
        o_v[pl.ds(g * 16, 16)] = 1.0 / (1.0 + jnp.exp(-v))
        return carry

    lax.fori_loop(0, _BPW // 16, sig, 0)
    pltpu.sync_copy(o_v, out_hbm.at[pl.ds(wid * _BPW, _BPW)])


_CT = (((1,), (1,)), ((), ()))


def _tc_dense1(x, W1le, e1, W1r, b1):
    g = 10
    bn = _N // g

    def body(x_ref, wle_ref, e1_ref, wr_ref, b1_ref, yext_ref, r1_ref):
        xb = x_ref[...]
        yext_ref[...] = lax.dot_general(
            xb, wle_ref[...], _CT, preferred_element_type=jnp.float32) + e1_ref[...]
        r1_ref[...] = lax.dot_general(
            xb, wr_ref[...], _CT, preferred_element_type=jnp.float32) + b1_ref[...]

    return pl.pallas_call(
        body,
        grid=(g,),
        in_specs=[pl.BlockSpec((bn, _D), lambda i: (i, 0)),
                  pl.BlockSpec((_W1, _D), lambda i: (0, 0)),
                  pl.BlockSpec((1, _W1), lambda i: (0, 0)),
                  pl.BlockSpec((_H, _D), lambda i: (0, 0)),
                  pl.BlockSpec((1, _H), lambda i: (0, 0))],
        out_specs=[pl.BlockSpec((bn, _W1), lambda i: (i, 0)),
                   pl.BlockSpec((bn, _H), lambda i: (i, 0))],
        out_shape=[jax.ShapeDtypeStruct((_N, _W1), jnp.float32),
                   jax.ShapeDtypeStruct((_N, _H), jnp.float32)],
    )(x, W1le, e1, W1r, b1)


def _tc_dense2(agg1p, r1, W2l, b2, W2r):
    g = 10
    bn = _N // g

    def body(aggp_ref, r1_ref, wl_ref, b_ref, wr_ref, y2_ref, r2_ref, inv_ref):
        a = aggp_ref[...]
        agg = a[0] + a[1]
        inv = 1.0 / jnp.maximum(agg[:, _H:_H + 1], 1.0)
        h = jnp.maximum(agg[:, :_H] * inv + r1_ref[...], 0.0)
        y2_ref[...] = lax.dot_general(
            h, wl_ref[...], _CT, preferred_element_type=jnp.float32)
        r2_ref[...] = lax.dot_general(
            h, wr_ref[...], _CT, preferred_element_type=jnp.float32) + b_ref[...]
        inv_ref[...] = inv

    return pl.pallas_call(
        body,
        grid=(g,),
        in_specs=[pl.BlockSpec((_NC, bn, _W1), lambda i: (0, i, 0)),
                  pl.BlockSpec((bn, _H), lambda i: (i, 0)),
                  pl.BlockSpec((_H, _H), lambda i: (0, 0)),
                  pl.BlockSpec((1, _H), lambda i: (0, 0)),
                  pl.BlockSpec((_H, _H), lambda i: (0, 0))],
        out_specs=[pl.BlockSpec((bn, _H), lambda i: (i, 0)),
                   pl.BlockSpec((bn, _H), lambda i: (i, 0)),
                   pl.BlockSpec((bn, 1), lambda i: (i, 0))],
        out_shape=[jax.ShapeDtypeStruct((_N, _H), jnp.float32),
                   jax.ShapeDtypeStruct((_N, _H), jnp.float32),
                   jax.ShapeDtypeStruct((_N, 1), jnp.float32)],
    )(agg1p, r1, W2l, b2, W2r)


def _tc_dense3(agg2p, r2, inv):
    g = 10
    bn = _N // g

    def body(aggp_ref, r2_ref, inv_ref, z_ref):
        a = aggp_ref[...]
        z_ref[...] = jnp.maximum((a[0] + a[1]) * inv_ref[...] + r2_ref[...], 0.0)

    return pl.pallas_call(
        body,
        grid=(g,),
        in_specs=[pl.BlockSpec((_NC, bn, _H), lambda i: (0, i, 0)),
                  pl.BlockSpec((bn, _H), lambda i: (i, 0)),
                  pl.BlockSpec((bn, 1), lambda i: (i, 0))],
        out_specs=pl.BlockSpec((bn, _H), lambda i: (i, 0)),
        out_shape=jax.ShapeDtypeStruct((_N, _H), jnp.float32),
    )(agg2p, r2, inv)


def kernel(x, edge_index, src, dst, W1l, b1l, W1r, W2l, b2l, W2r):
    f32 = jnp.float32
    W1le = jnp.zeros((_W1, _D), f32).at[:_H].set(W1l)
    e1 = jnp.zeros((1, _W1), f32).at[0, _H].set(1.0)

    pad = _EP - _E
    s_r = jnp.concatenate(
        [edge_index[0], jnp.arange(pad, dtype=jnp.int32) % _N]
    ).reshape(_NW, _NCHK, _CH)
    d_r = jnp.concatenate(
        [edge_index[1], _N + jnp.arange(pad, dtype=jnp.int32) % (_NP - _N)]
    ).reshape(_NW, _NCHK, _CH)
    zeros1 = jnp.zeros((_NP, _W1), f32)
    zeros2 = jnp.zeros((_NP, _H), f32)

    yext, r1 = _tc_dense1(x, W1le, e1, W1r, b1l.reshape(1, _H))
    agg1p = _segsum80(s_r, d_r, yext, zeros1)
    y2, r2, inv = _tc_dense2(agg1p, r1, W2l, b2l.reshape(1, _H), W2r)
    agg2p = _segsum64(s_r, d_r, y2, zeros2)
    z = _tc_dense3(agg2p, r2, inv)

    return _decode(src.reshape(_NW, 2, 128), dst.reshape(_NW, 2, 128), z)
